# baseline (speedup 1.0000x reference)
import numpy as np

import jax
import jax.numpy as jnp
from jax import lax
from jax.experimental import pallas as pl
from jax.experimental.pallas import tpu as pltpu
from jax.experimental.pallas import tpu_sc as plsc

H = 128
N_NODES = 10000
N_EDGES = 320000

NC = 2
NS = 16
NW = NC * NS
L = 16

DW = 8
E_CHUNK = 40
EDGES_PER_W = N_EDGES // NW
N_CHUNKS = EDGES_PER_W // E_CHUNK
N_PAD = 10240
ROWS_PER_TILE = N_PAD // NS
ZCOPIES = ROWS_PER_TILE // E_CHUNK

_HIGH = jax.lax.Precision.HIGHEST

_PERM = np.empty((H,), np.int32)
for _j2 in range(H // 32):
    for _i in range(L):
        _PERM[32 * _j2 + 2 * _i] = 32 * _j2 + _i
        _PERM[32 * _j2 + 2 * _i + 1] = 32 * _j2 + L + _i


def _gelu_sc(x):
    y = jnp.clip(x, -4.0, 4.0)
    u = y * y
    pphi = (((((((-1.5809654e-09 * u + 1.2172114e-07) * u - 4.1010894e-06)
                * u + 8.066989e-05) * u - 1.0482192e-03) * u
              + 9.664918e-03) * u - 6.617544e-02) * u + 3.9884752e-01)
    return x * (0.5 + y * pphi)


def _stage1_body(h_ref, wa_ref, wb_ref, b1_ref, a_ref, b_ref):
    h = h_ref[...]
    a_ref[...] = jnp.dot(h, wa_ref[...], precision=_HIGH) + b1_ref[...]
    b_ref[...] = jnp.dot(h, wb_ref[...], precision=_HIGH)


def _stage1(hidden, w1a, w1b, b1e):
    blk = 1000
    grid = (N_NODES // blk,)
    return pl.pallas_call(
        _stage1_body,
        grid=grid,
        in_specs=[
            pl.BlockSpec((blk, H), lambda i: (i, 0)),
            pl.BlockSpec((H, H), lambda i: (0, 0)),
            pl.BlockSpec((H, H), lambda i: (0, 0)),
            pl.BlockSpec((1, H), lambda i: (0, 0)),
        ],
        out_specs=[
            pl.BlockSpec((blk, H), lambda i: (i, 0)),
            pl.BlockSpec((blk, H), lambda i: (i, 0)),
        ],
        out_shape=[
            jax.ShapeDtypeStruct((N_NODES, H), jnp.float32),
            jax.ShapeDtypeStruct((N_NODES, H), jnp.float32),
        ],
    )(hidden, w1a, w1b, b1e)


def _sc_body(a_hbm, b_hbm, idx_hbm, wrow_hbm, z128_hbm,
             z8_hbm, ones8_hbm, g_hbm, d_hbm, idxr, a2, b2, g2,
             ones_v, wr_v, ewb, acc_sh, deg_sh, si, sg, ss):
    cid = lax.axis_index("core")
    sid = lax.axis_index("subcore")
    wid = sid * NC + cid
    tid = sid

    @pl.loop(0, ZCOPIES)
    def _(c):
        rows = pl.ds(tid * ROWS_PER_TILE + c * E_CHUNK, E_CHUNK)
        pltpu.sync_copy(z128_hbm, acc_sh.at[rows])
        pltpu.sync_copy(z8_hbm, deg_sh.at[rows])

    pltpu.sync_copy(ones8_hbm, ones_v)
    pltpu.sync_copy(wrow_hbm, wr_v)
    plsc.subcore_barrier()

    wrjs = [wr_v[pl.ds(j * L, L)] for j in range(H // L)]

    def idx_copy(k, q):
        return pltpu.make_async_copy(idx_hbm.at[wid, k], idxr.at[q],
                                     si.at[q])

    def gather_copies(k3, q):
        return (
            pltpu.make_async_copy(a_hbm.at[idxr.at[q, pl.ds(0, E_CHUNK)]],
                                  a2.at[k3], sg.at[k3]),
            pltpu.make_async_copy(
                b_hbm.at[idxr.at[q, pl.ds(E_CHUNK, E_CHUNK)]],
                b2.at[k3], sg.at[k3]),
        )

    def scatter_copies(b, q):
        dref = idxr.at[q, pl.ds(E_CHUNK, E_CHUNK)]
        return (
            pltpu.make_async_copy(g2.at[b], acc_sh.at[dref], ss.at[b]),
            pltpu.make_async_copy(ones_v, deg_sh.at[dref], ss.at[b]),
        )

    for k in (0, 1, 2):
        idx_copy(k, k).start()
    for k in (0, 1):
        idx_copy(k, k).wait()
        for c in gather_copies(k, k):
            c.start()

    @pl.loop(0, N_CHUNKS)
    def _(k):
        q = lax.rem(k, 8)
        b3 = lax.rem(k, 3)
        b = lax.rem(k, 2)
        qn = lax.rem(k + 2, 8)
        qi = lax.rem(k + 3, 8)

        for c in gather_copies(b3, q):
            c.wait()

        @pl.when(k >= 2)
        def _():
            for c in scatter_copies(b, qn):
                c.wait()

        @pl.when(k + 3 < N_CHUNKS)
        def _():
            idx_copy(k + 3, qi).start()

        @pl.when(k + 2 < N_CHUNKS)
        def _():
            idx_copy(k + 2, qn).wait()
            for c in gather_copies(lax.rem(k + 2, 3), qn):
                c.start()

        for e0, nk in ((0, L), (16, L), (32, 8)):
            wv = plsc.bitcast(idxr[q, pl.ds(2 * E_CHUNK + e0, L)],
                              jnp.float32)
            for kk in range(nk):
                ewb[e0 + kk, :] = jnp.full((L,), wv[kk], jnp.float32)

        @plsc.parallel_loop(0, E_CHUNK, unroll=2)
        def _(e):
            w16 = ewb[e, :]
            for j in range(H // L):
                sl = pl.ds(j * L, L)
                x = a2[b3, e, sl] + b2[b3, e, sl] + w16 * wrjs[j]
                g2[b, e, sl] = _gelu_sc(x)

        sca, scd = scatter_copies(b, q)
        sca.start(add=True)
        scd.start(add=True)

    for k in (N_CHUNKS - 2, N_CHUNKS - 1):
        for c in scatter_copies(k % 2, k % 8):
            c.wait()

    plsc.subcore_barrier()

    rows = pl.ds(tid * ROWS_PER_TILE, ROWS_PER_TILE)
    pltpu.sync_copy(acc_sh.at[rows], g_hbm.at[cid, rows])
    pltpu.sync_copy(deg_sh.at[rows], d_hbm.at[cid, rows])


def _stage2(a_tab, b_tab, src, dst, ew, wrow):
    mesh = plsc.VectorSubcoreMesh(core_axis_name="core",
                                  subcore_axis_name="subcore")
    kern = pl.kernel(
        _sc_body,
        out_type=[
            jax.ShapeDtypeStruct((NC, N_PAD, H), jnp.float32),
            jax.ShapeDtypeStruct((NC, N_PAD, DW), jnp.float32),
        ],
        mesh=mesh,
        scratch_types=[
            pltpu.VMEM((8, 128), jnp.int32),
            pltpu.VMEM((3, E_CHUNK, H), jnp.float32),
            pltpu.VMEM((3, E_CHUNK, H), jnp.float32),
            pltpu.VMEM((2, E_CHUNK, H), jnp.float32),
            pltpu.VMEM((E_CHUNK, DW), jnp.float32),
            pltpu.VMEM((H,), jnp.float32),
            pltpu.VMEM((E_CHUNK, L), jnp.float32),
            pltpu.VMEM_SHARED((N_PAD, H), jnp.float32),
            pltpu.VMEM_SHARED((N_PAD, DW), jnp.float32),
            pltpu.SemaphoreType.DMA((8,)),
            pltpu.SemaphoreType.DMA((3,)),
            pltpu.SemaphoreType.DMA((2,)),
        ],
        compiler_params=pltpu.CompilerParams(use_tc_tiling_on_sc=False,
                                            needs_layout_passes=False),
    )
    src_i = src.reshape(NW, N_CHUNKS, E_CHUNK)
    dst_i = dst.reshape(NW, N_CHUNKS, E_CHUNK)
    ew_i = lax.bitcast_convert_type(ew, jnp.int32).reshape(
        NW, N_CHUNKS, E_CHUNK)
    pad = jnp.zeros((NW, N_CHUNKS, 128 - 3 * E_CHUNK), jnp.int32)
    idx_packed = jnp.concatenate([src_i, dst_i, ew_i, pad], axis=2)
    z128 = jnp.zeros((E_CHUNK, H), jnp.float32)
    z8 = jnp.zeros((E_CHUNK, DW), jnp.float32)
    ones8 = jnp.ones((E_CHUNK, DW), jnp.float32)
    return kern(a_tab, b_tab, idx_packed, wrow, z128, z8, ones8)


def _stage3_body(h_ref, g_ref, d_ref, w2e_ref, b2e_ref, w1h_ref, w1a_ref,
                 b1u_ref, w2u_ref, b2u_ref, gam_ref, bet_ref, o_ref):
    h = h_ref[...]
    g = g_ref[0] + g_ref[1]
    deg = d_ref[0, :, :1] + d_ref[1, :, :1]
    agg = jnp.dot(g, w2e_ref[...], precision=_HIGH) + deg * b2e_ref[...]
    pre = (jnp.dot(h, w1h_ref[...], precision=_HIGH)
           + jnp.dot(agg, w1a_ref[...], precision=_HIGH) + b1u_ref[...])
    act = 0.5 * pre * (1.0 + lax.erf(pre * 0.7071067811865476))
    upd = jnp.dot(act, w2u_ref[...], precision=_HIGH) + b2u_ref[...]
    x = h + upd
    mu = jnp.mean(x, axis=-1, keepdims=True)
    var = jnp.mean((x - mu) ** 2, axis=-1, keepdims=True)
    o_ref[...] = (x - mu) / jnp.sqrt(var + 1e-5) * gam_ref[...] + bet_ref[...]


def _stage3(hidden, g, d, w2e, b2e, w1h, w1a, b1u, w2u, b2u, gamma, beta):
    blk = 1000
    grid = (N_NODES // blk,)
    full = lambda i: (0, 0)
    return pl.pallas_call(
        _stage3_body,
        grid=grid,
        in_specs=[
            pl.BlockSpec((blk, H), lambda i: (i, 0)),
            pl.BlockSpec((NC, blk, H), lambda i: (0, i, 0)),
            pl.BlockSpec((NC, blk, DW), lambda i: (0, i, 0)),
            pl.BlockSpec((H, H), full),
            pl.BlockSpec((1, H), full),
            pl.BlockSpec((H, H), full),
            pl.BlockSpec((H, H), full),
            pl.BlockSpec((1, H), full),
            pl.BlockSpec((H, H), full),
            pl.BlockSpec((1, H), full),
            pl.BlockSpec((1, H), full),
            pl.BlockSpec((1, H), full),
        ],
        out_specs=pl.BlockSpec((blk, H), lambda i: (i, 0)),
        out_shape=jax.ShapeDtypeStruct((N_NODES, H), jnp.float32),
    )(hidden, g, d, w2e, b2e, w1h, w1a, b1u, w2u, b2u, gamma, beta)


def kernel(hidden, edge_index, edge_weight, W1e, b1e, W2e, b2e,
           W1u, b1u, W2u, b2u, gamma, beta):
    src = edge_index[0].astype(jnp.int32)
    dst = edge_index[1].astype(jnp.int32)
    ew = edge_weight.astype(jnp.float32)

    w1a = W1e[:H]
    w1b = W1e[H:2 * H]
    wrow = W1e[2 * H]

    a_tab, b_tab = _stage1(hidden, w1a, w1b, b1e.reshape(1, H))
    g, d = _stage2(a_tab, b_tab, src, dst, ew, wrow)
    return _stage3(hidden, g, d, W2e, b2e.reshape(1, H),
                   W1u[:H], W1u[H:], b1u.reshape(1, H),
                   W2u, b2u.reshape(1, H),
                   gamma.reshape(1, H), beta.reshape(1, H))


# speedup vs baseline: 1.0653x; 1.0653x over previous
import numpy as np

import jax
import jax.numpy as jnp
from jax import lax
from jax.experimental import pallas as pl
from jax.experimental.pallas import tpu as pltpu
from jax.experimental.pallas import tpu_sc as plsc

H = 128
N_NODES = 10000
N_EDGES = 320000

NC = 2
NS = 16
NW = NC * NS
L = 16

GW = 144
E_CHUNK = 40
EDGES_PER_W = N_EDGES // NW
N_CHUNKS = EDGES_PER_W // E_CHUNK
N_PAD = 10240
ROWS_PER_TILE = N_PAD // NS
ZCOPIES = ROWS_PER_TILE // E_CHUNK
DCOL = H

_HIGH = jax.lax.Precision.HIGHEST

_PERM = np.empty((H,), np.int32)
for _j2 in range(H // 32):
    for _i in range(L):
        _PERM[32 * _j2 + 2 * _i] = 32 * _j2 + _i
        _PERM[32 * _j2 + 2 * _i + 1] = 32 * _j2 + L + _i


def _gelu_sc(x):
    y = jnp.clip(x, -4.0, 4.0)
    u = y * y
    pphi = (((((((-1.5809654e-09 * u + 1.2172114e-07) * u - 4.1010894e-06)
                * u + 8.066989e-05) * u - 1.0482192e-03) * u
              + 9.664918e-03) * u - 6.617544e-02) * u + 3.9884752e-01)
    return x * (0.5 + y * pphi)


def _stage1_body(h_ref, wa_ref, wb_ref, b1_ref, a_ref, b_ref):
    h = h_ref[...]
    a_ref[...] = jnp.dot(h, wa_ref[...], precision=_HIGH) + b1_ref[...]
    b_ref[...] = jnp.dot(h, wb_ref[...], precision=_HIGH)


def _stage1(hidden, w1a, w1b, b1e):
    blk = 1000
    grid = (N_NODES // blk,)
    return pl.pallas_call(
        _stage1_body,
        grid=grid,
        in_specs=[
            pl.BlockSpec((blk, H), lambda i: (i, 0)),
            pl.BlockSpec((H, H), lambda i: (0, 0)),
            pl.BlockSpec((H, H), lambda i: (0, 0)),
            pl.BlockSpec((1, H), lambda i: (0, 0)),
        ],
        out_specs=[
            pl.BlockSpec((blk, H), lambda i: (i, 0)),
            pl.BlockSpec((blk, H), lambda i: (i, 0)),
        ],
        out_shape=[
            jax.ShapeDtypeStruct((N_NODES, H), jnp.float32),
            jax.ShapeDtypeStruct((N_NODES, H), jnp.float32),
        ],
    )(hidden, w1a, w1b, b1e)


def _sc_body(a_hbm, b_hbm, src_hbm, dst_hbm, ew_hbm, wrow_hbm, z144_hbm,
             g_hbm, srcs, dsts, ews, a2, b2, g2, wr_v, ewb,
             acc_sh, si, sg, ss):
    cid = lax.axis_index("core")
    sid = lax.axis_index("subcore")
    wid = sid * NC + cid
    tid = sid

    @pl.loop(0, ZCOPIES)
    def _(c):
        rows = pl.ds(tid * ROWS_PER_TILE + c * E_CHUNK, E_CHUNK)
        pltpu.sync_copy(z144_hbm, acc_sh.at[rows])

    pltpu.sync_copy(wrow_hbm, wr_v)

    e0vec = jnp.where(lax.iota(jnp.int32, L) == 0, 1.0, 0.0)

    @pl.loop(0, E_CHUNK)
    def _(e):
        g2[0, e, pl.ds(DCOL, L)] = e0vec
        g2[1, e, pl.ds(DCOL, L)] = e0vec

    plsc.subcore_barrier()

    wrjs = [wr_v[pl.ds(j * L, L)] for j in range(H // L)]

    def idx_copies(k, p):
        base = pl.ds(k * E_CHUNK, E_CHUNK)
        return (
            pltpu.make_async_copy(src_hbm.at[wid, base], srcs.at[p],
                                  si.at[p]),
            pltpu.make_async_copy(dst_hbm.at[wid, base], dsts.at[p],
                                  si.at[p]),
            pltpu.make_async_copy(ew_hbm.at[wid, base],
                                  ews.at[p, pl.ds(0, E_CHUNK)], si.at[p]),
        )

    def gather_copies(b, p):
        return (
            pltpu.make_async_copy(a_hbm.at[srcs.at[p]], a2.at[b], sg.at[b]),
            pltpu.make_async_copy(b_hbm.at[dsts.at[p]], b2.at[b], sg.at[b]),
        )

    def scatter_copy(b, p):
        return pltpu.make_async_copy(g2.at[b], acc_sh.at[dsts.at[p]],
                                     ss.at[b])

    for k in (0, 1):
        for c in idx_copies(k, k):
            c.start()
    for k in (0, 1):
        for c in idx_copies(k, k):
            c.wait()
        for c in gather_copies(k, k):
            c.start()

    @pl.loop(0, N_CHUNKS)
    def _(k):
        b = lax.rem(k, 2)
        p = lax.rem(k, 4)
        p2 = lax.rem(k + 2, 4)

        for c in gather_copies(b, p):
            c.wait()

        @pl.when(k >= 2)
        def _():
            scatter_copy(b, p2).wait()

        @pl.when(k + 2 < N_CHUNKS)
        def _():
            for c in idx_copies(k + 2, p2):
                c.start()

        for e0, nk in ((0, L), (16, L), (32, 8)):
            wv = ews[p, pl.ds(e0, L)]
            for kk in range(nk):
                ewb[e0 + kk, :] = jnp.full((L,), wv[kk], jnp.float32)

        @plsc.parallel_loop(0, E_CHUNK, unroll=2)
        def _(e):
            w16 = ewb[e, :]
            for j in range(H // L):
                sl = pl.ds(j * L, L)
                x = a2[b, e, sl] + b2[b, e, sl] + w16 * wrjs[j]
                g2[b, e, sl] = _gelu_sc(x)

        scatter_copy(b, p).start(add=True)

        @pl.when(k + 2 < N_CHUNKS)
        def _():
            for c in idx_copies(k + 2, p2):
                c.wait()
            for c in gather_copies(b, p2):
                c.start()

    for k in (N_CHUNKS - 2, N_CHUNKS - 1):
        scatter_copy(k % 2, k % 4).wait()

    plsc.subcore_barrier()

    rows = pl.ds(tid * ROWS_PER_TILE, ROWS_PER_TILE)
    pltpu.sync_copy(acc_sh.at[rows], g_hbm.at[cid, rows])


def _stage2(a_tab, b_tab, src, dst, ew, wrow):
    mesh = plsc.VectorSubcoreMesh(core_axis_name="core",
                                  subcore_axis_name="subcore")
    kern = pl.kernel(
        _sc_body,
        out_type=jax.ShapeDtypeStruct((NC, N_PAD, GW), jnp.float32),
        mesh=mesh,
        scratch_types=[
            pltpu.VMEM((4, E_CHUNK), jnp.int32),
            pltpu.VMEM((4, E_CHUNK), jnp.int32),
            pltpu.VMEM((4, E_CHUNK + 8), jnp.float32),
            pltpu.VMEM((2, E_CHUNK, H), jnp.float32),
            pltpu.VMEM((2, E_CHUNK, H), jnp.float32),
            pltpu.VMEM((2, E_CHUNK, GW), jnp.float32),
            pltpu.VMEM((H,), jnp.float32),
            pltpu.VMEM((E_CHUNK, L), jnp.float32),
            pltpu.VMEM_SHARED((N_PAD, GW), jnp.float32),
            pltpu.SemaphoreType.DMA((4,)),
            pltpu.SemaphoreType.DMA((2,)),
            pltpu.SemaphoreType.DMA((2,)),
        ],
        compiler_params=pltpu.CompilerParams(use_tc_tiling_on_sc=False),
    )
    srcr = src.reshape(NW, EDGES_PER_W)
    dstr = dst.reshape(NW, EDGES_PER_W)
    ewr = ew.reshape(NW, EDGES_PER_W)
    z144 = jnp.zeros((E_CHUNK, GW), jnp.float32)
    return kern(a_tab, b_tab, srcr, dstr, ewr, wrow, z144)


def _stage3_body(h_ref, g_ref, w2e_ref, b2e_ref, w1h_ref, w1a_ref,
                 b1u_ref, w2u_ref, b2u_ref, gam_ref, bet_ref, o_ref):
    h = h_ref[...]
    gfull = g_ref[0] + g_ref[1]
    g = gfull[:, :H]
    deg = gfull[:, DCOL:DCOL + 1]
    agg = jnp.dot(g, w2e_ref[...], precision=_HIGH) + deg * b2e_ref[...]
    pre = (jnp.dot(h, w1h_ref[...], precision=_HIGH)
           + jnp.dot(agg, w1a_ref[...], precision=_HIGH) + b1u_ref[...])
    act = 0.5 * pre * (1.0 + lax.erf(pre * 0.7071067811865476))
    upd = jnp.dot(act, w2u_ref[...], precision=_HIGH) + b2u_ref[...]
    x = h + upd
    mu = jnp.mean(x, axis=-1, keepdims=True)
    var = jnp.mean((x - mu) ** 2, axis=-1, keepdims=True)
    o_ref[...] = (x - mu) / jnp.sqrt(var + 1e-5) * gam_ref[...] + bet_ref[...]


def _stage3(hidden, g, w2e, b2e, w1h, w1a, b1u, w2u, b2u, gamma, beta):
    blk = 1000
    grid = (N_NODES // blk,)
    full = lambda i: (0, 0)
    return pl.pallas_call(
        _stage3_body,
        grid=grid,
        in_specs=[
            pl.BlockSpec((blk, H), lambda i: (i, 0)),
            pl.BlockSpec((NC, blk, GW), lambda i: (0, i, 0)),
            pl.BlockSpec((H, H), full),
            pl.BlockSpec((1, H), full),
            pl.BlockSpec((H, H), full),
            pl.BlockSpec((H, H), full),
            pl.BlockSpec((1, H), full),
            pl.BlockSpec((H, H), full),
            pl.BlockSpec((1, H), full),
            pl.BlockSpec((1, H), full),
            pl.BlockSpec((1, H), full),
        ],
        out_specs=pl.BlockSpec((blk, H), lambda i: (i, 0)),
        out_shape=jax.ShapeDtypeStruct((N_NODES, H), jnp.float32),
    )(hidden, g, w2e, b2e, w1h, w1a, b1u, w2u, b2u, gamma, beta)


def kernel(hidden, edge_index, edge_weight, W1e, b1e, W2e, b2e,
           W1u, b1u, W2u, b2u, gamma, beta):
    src = edge_index[0].astype(jnp.int32)
    dst = edge_index[1].astype(jnp.int32)
    ew = edge_weight.astype(jnp.float32)

    w1a = W1e[:H]
    w1b = W1e[H:2 * H]
    wrow = W1e[2 * H]

    a_tab, b_tab = _stage1(hidden, w1a, w1b, b1e.reshape(1, H))
    g = _stage2(a_tab, b_tab, src, dst, ew, wrow)
    return _stage3(hidden, g, W2e, b2e.reshape(1, H),
                   W1u[:H], W1u[H:], b1u.reshape(1, H),
                   W2u, b2u.reshape(1, H),
                   gamma.reshape(1, H), beta.reshape(1, H))
